# baseline (speedup 1.0000x reference)
import functools

import jax
import jax.numpy as jnp
from jax import lax
from jax.experimental import pallas as pl
from jax.experimental.pallas import tpu as pltpu
from jax.experimental.pallas import tpu_sc as plsc

VOCAB = 1000
D = 64
BATCH = 4096
SEQ = 200
VPAD = 1024

NC = 2
NS = 16
NW = NC * NS
DG = 16
BG = 512
NJ = BG // 16
CSTRIDE = DG * VPAD + 16
TABW = 4 * CSTRIDE


def _body(tab_hbm, idx_hbm, out_hbm, tab_v, idx0, idx1, buf0, buf1,
          osem0, osem1, isem0, isem1):
    wid = lax.axis_index("s") * NC + lax.axis_index("c")
    dgrp = lax.rem(wid, 4)
    bgrp = wid // 4
    d0 = dgrp * DG
    b0 = bgrp * BG

    pltpu.sync_copy(tab_hbm.at[pl.ds(dgrp * TABW, TABW)], tab_v)
    bvec = lax.rem(lax.iota(jnp.int32, 16), 4) * (CSTRIDE + 1)

    rings = (idx0, idx1)
    isems = (isem0, isem1)
    bufs = (buf0, buf1)
    osems = (osem0, osem1)

    def idx_src(s):
        return idx_hbm.at[pl.ds(bgrp * (SEQ * BG) + s * BG, BG)]

    def start_idx(s, r):
        pltpu.async_copy(idx_src(s), rings[r], isems[r])

    def wait_idx(s, r):
        pltpu.make_async_copy(idx_src(s), rings[r], isems[r]).wait()

    def fill_half(buf, ring, h):
        for j in range(h * NJ // 2, (h + 1) * NJ // 2):
            idxv = ring[pl.ds(j * 16, 16)] + bvec

            @plsc.parallel_loop(0, DG, unroll=16)
            def _(d):
                col = plsc.load_gather(tab_v, [idxv + d * VPAD])
                buf[d, pl.ds(j * 16, 16)] = col

    HB = BG // 2

    def out_half(s, b, h):
        return pltpu.make_async_copy(
            bufs[b].at[:, pl.ds(h * HB, HB)],
            out_hbm.at[s, pl.ds(d0, DG), pl.ds(b0 + h * HB, HB)],
            osems[b],
        )

    def fill_and_send(s, b):
        fill_half(bufs[b], rings[b], 0)
        out_half(s, b, 0).start()
        fill_half(bufs[b], rings[b], 1)
        out_half(s, b, 1).start()

    def wait_out(s, b):
        out_half(s, b, 0).wait()
        out_half(s, b, 1).wait()

    start_idx(0, 0)
    start_idx(1, 1)
    for s in range(2):
        wait_idx(s, s)
        fill_and_send(s, s)
        start_idx(s + 2, s)

    def step(i, _):
        for b in range(2):
            s = 2 + 2 * i + b
            wait_out(s - 2, b)
            wait_idx(s, b)
            fill_and_send(s, b)

            @pl.when(s + 2 < SEQ)
            def _():
                start_idx(s + 2, b)

        return 0

    lax.fori_loop(0, (SEQ - 2) // 2, step, 0, unroll=False)
    wait_out(SEQ - 2, 0)
    wait_out(SEQ - 1, 1)


def _lookup(tab4, idxf):
    mesh = plsc.VectorSubcoreMesh(core_axis_name="c", subcore_axis_name="s")
    f = functools.partial(
        pl.kernel,
        mesh=mesh,
        out_type=jax.ShapeDtypeStruct((SEQ, D, BATCH), jnp.float32),
        scratch_types=[
            pltpu.VMEM((TABW,), jnp.float32),
            pltpu.VMEM((BG,), jnp.int32),
            pltpu.VMEM((BG,), jnp.int32),
            pltpu.VMEM((DG, BG), jnp.float32),
            pltpu.VMEM((DG, BG), jnp.float32),
            pltpu.SemaphoreType.DMA,
            pltpu.SemaphoreType.DMA,
            pltpu.SemaphoreType.DMA,
            pltpu.SemaphoreType.DMA,
        ],
        compiler_params=pltpu.CompilerParams(
            use_tc_tiling_on_sc=True, needs_layout_passes=False
        ),
    )(_body)
    return f(tab4, idxf)


@jax.jit
def kernel(word_sequences, embedding_table):
    tabT = (
        jnp.zeros((D, VPAD), jnp.float32)
        .at[:, :VOCAB]
        .set(embedding_table.astype(jnp.float32).T)
    )
    slices = tabT.reshape(4, DG * VPAD)
    tab4 = jnp.stack(
        [jnp.pad(slices, ((0, 0), (c, 16 - c))) for c in range(4)],
        axis=1,
    ).reshape(-1)
    idxf = (
        word_sequences.astype(jnp.int32)
        .reshape(8, BG, SEQ)
        .transpose(0, 2, 1)
        .reshape(-1)
    )
    out = _lookup(tab4, idxf)
    return out.transpose(2, 0, 1)


# speedup vs baseline: 1.0133x; 1.0133x over previous
import functools

import jax
import jax.numpy as jnp
from jax import lax
from jax.experimental import pallas as pl
from jax.experimental.pallas import tpu as pltpu
from jax.experimental.pallas import tpu_sc as plsc

VOCAB = 1000
D = 64
BATCH = 4096
SEQ = 200
VPAD = 1024

NC = 2
NS = 16
NW = NC * NS
DG = 16
BG = 512
NJ = BG // 16
CSTRIDE = DG * VPAD + 16
TABW = 4 * CSTRIDE


def _body(tab_hbm, idx_hbm, out_hbm, tab_v, idx0, idx1, buf0, buf1,
          osem0, osem1, isem0, isem1):
    wid = lax.axis_index("s") * NC + lax.axis_index("c")
    dgrp = lax.rem(wid, 4)
    bgrp = wid // 4
    d0 = dgrp * DG
    b0 = bgrp * BG

    pltpu.sync_copy(tab_hbm.at[pl.ds(dgrp * TABW, TABW)], tab_v)
    bvec = lax.rem(lax.iota(jnp.int32, 16), 4) * (CSTRIDE + 1)

    rings = (idx0, idx1)
    isems = (isem0, isem1)
    bufs = (buf0, buf1)
    osems = (osem0, osem1)

    def idx_src(s):
        return idx_hbm.at[pl.ds(bgrp * (SEQ * BG) + s * BG, BG)]

    def start_idx(s, r):
        pltpu.async_copy(idx_src(s), rings[r], isems[r])

    def wait_idx(s, r):
        pltpu.make_async_copy(idx_src(s), rings[r], isems[r]).wait()

    def fill(buf, ring):
        for j in range(NJ):
            idxv = ring[pl.ds(j * 16, 16)] + bvec

            @plsc.parallel_loop(0, DG, unroll=16)
            def _(d):
                col = plsc.load_gather(tab_v, [idxv + d * VPAD])
                buf[d, pl.ds(j * 16, 16)] = col

    def out_dst(s):
        return out_hbm.at[s, pl.ds(d0, DG), pl.ds(b0, BG)]

    def start_out(s, b):
        pltpu.async_copy(bufs[b], out_dst(s), osems[b])

    def wait_out(s, b):
        pltpu.make_async_copy(bufs[b], out_dst(s), osems[b]).wait()

    start_idx(0, 0)
    start_idx(1, 1)
    for s in range(2):
        wait_idx(s, s)
        fill(bufs[s], rings[s])
        start_idx(s + 2, s)
        start_out(s, s)

    def step(i, _):
        for b in range(2):
            s = 2 + 2 * i + b
            wait_out(s - 2, b)
            wait_idx(s, b)
            fill(bufs[b], rings[b])

            @pl.when(s + 2 < SEQ)
            def _():
                start_idx(s + 2, b)

            start_out(s, b)
        return 0

    lax.fori_loop(0, (SEQ - 2) // 2, step, 0, unroll=False)
    wait_out(SEQ - 2, 0)
    wait_out(SEQ - 1, 1)


def _lookup(tab4, idxf):
    mesh = plsc.VectorSubcoreMesh(core_axis_name="c", subcore_axis_name="s")
    f = functools.partial(
        pl.kernel,
        mesh=mesh,
        out_type=jax.ShapeDtypeStruct((SEQ, D, BATCH), jnp.float32),
        scratch_types=[
            pltpu.VMEM((TABW,), jnp.float32),
            pltpu.VMEM((BG,), jnp.int32),
            pltpu.VMEM((BG,), jnp.int32),
            pltpu.VMEM((DG, BG), jnp.float32),
            pltpu.VMEM((DG, BG), jnp.float32),
            pltpu.SemaphoreType.DMA,
            pltpu.SemaphoreType.DMA,
            pltpu.SemaphoreType.DMA,
            pltpu.SemaphoreType.DMA,
        ],
        compiler_params=pltpu.CompilerParams(
            use_tc_tiling_on_sc=True, needs_layout_passes=False
        ),
    )(_body)
    return f(tab4, idxf)


@jax.jit
def kernel(word_sequences, embedding_table):
    tabT = (
        jnp.zeros((D, VPAD), jnp.float32)
        .at[:, :VOCAB]
        .set(embedding_table.astype(jnp.float32).T)
    )
    slices = tabT.reshape(4, DG * VPAD)
    tab4 = jnp.stack(
        [jnp.pad(slices, ((0, 0), (c, 16 - c))) for c in range(4)],
        axis=1,
    ).reshape(-1)
    idxf = (
        word_sequences.astype(jnp.int32)
        .reshape(8, BG, SEQ)
        .transpose(0, 2, 1)
        .reshape(-1)
    )
    out = _lookup(tab4, idxf)
    return out.transpose(2, 0, 1)
